# baseline (speedup 1.0000x reference)
import functools

import jax
import jax.numpy as jnp
from jax import lax
from jax.experimental import pallas as pl
from jax.experimental.pallas import tpu as pltpu
from jax.experimental.pallas import tpu_sc as plsc

NSIZE = 1000000
EM = 64
B = 4096
NNEG = 20

NW = 32
RL = B * NNEG + B
PER_W = RL // NW
CHUNK = 2688
NCH = PER_W // CHUNK

PS = 65536
PGRID = (NSIZE + PS - 1) // PS

BE = B
NEG_BLKS = B * NNEG // BE


def _project_body(tblT, w0, w1, o0, o1, o2, o3):
    wmat = jnp.concatenate(
        [w0[0:EM, :], w1[0:EM, :], w0[EM:, :], w1[EM:, :]], axis=1)
    P = lax.dot_general(wmat, tblT[...], (((0,), (0,)), ((), ())),
                        preferred_element_type=jnp.float32)
    o0[...] = P[0, :]
    o1[...] = P[1, :]
    o2[...] = P[2, :]
    o3[...] = P[3, :]


def _project_tc(tableT, W0, W1):
    out_shapes = tuple(
        jax.ShapeDtypeStruct((NSIZE,), jnp.float32) for _ in range(4))
    return pl.pallas_call(
        _project_body,
        grid=(PGRID,),
        in_specs=[
            pl.BlockSpec((EM, PS), lambda i: (0, i)),
            pl.BlockSpec((2 * EM, 1), lambda i: (0, 0)),
            pl.BlockSpec((2 * EM, 1), lambda i: (0, 0)),
        ],
        out_specs=tuple(pl.BlockSpec((PS,), lambda i: (i,))
                        for _ in range(4)),
        out_shape=out_shapes,
    )(tableT, W0, W1)


def _gather_proj_sc(p0, p1, p2, p3, idx_l, idx_r):
    mesh = plsc.VectorSubcoreMesh(core_axis_name="c", subcore_axis_name="s")

    @functools.partial(
        pl.kernel,
        out_type=tuple(
            jax.ShapeDtypeStruct((1, RL), jnp.float32) for _ in range(4)),
        mesh=mesh,
        compiler_params=pltpu.CompilerParams(use_tc_tiling_on_sc=False),
        scratch_types=[
            pltpu.VMEM((PER_W,), jnp.int32),
            pltpu.VMEM((PER_W,), jnp.int32),
            pltpu.VMEM((CHUNK,), jnp.float32),
            pltpu.VMEM((CHUNK,), jnp.float32),
            pltpu.VMEM((CHUNK,), jnp.float32),
            pltpu.VMEM((CHUNK,), jnp.float32),
            pltpu.SemaphoreType.DMA,
        ],
    )
    def gather_kernel(p0_h, p1_h, p2_h, p3_h, il_h, ir_h,
                      o0_h, o1_h, o2_h, o3_h,
                      il_v, ir_v, s0, s1, s2, s3, sem):
        wid = lax.axis_index("s") * 2 + lax.axis_index("c")
        base = wid * PER_W
        pltpu.sync_copy(il_h.at[pl.ds(base, PER_W)], il_v)
        pltpu.sync_copy(ir_h.at[pl.ds(base, PER_W)], ir_v)
        for c in range(NCH):
            il_c = il_v.at[pl.ds(c * CHUNK, CHUNK)]
            ir_c = ir_v.at[pl.ds(c * CHUNK, CHUNK)]
            cps = [
                pltpu.async_copy(p0_h.at[il_c], s0, sem),
                pltpu.async_copy(p1_h.at[il_c], s1, sem),
                pltpu.async_copy(p2_h.at[ir_c], s2, sem),
                pltpu.async_copy(p3_h.at[ir_c], s3, sem),
            ]
            for cp in cps:
                cp.wait()
            dst = pl.ds(base + c * CHUNK, CHUNK)
            pltpu.sync_copy(s0, o0_h.at[0, dst])
            pltpu.sync_copy(s1, o1_h.at[0, dst])
            pltpu.sync_copy(s2, o2_h.at[0, dst])
            pltpu.sync_copy(s3, o3_h.at[0, dst])

    return gather_kernel(p0, p1, p2, p3, idx_l, idx_r)


def _score_body(nA0, nA1, nB0, nB1, eA0, eA1, eB0, eB1, kd, prm,
                inten_o, surv_o):
    n = pl.program_id(0)
    b0 = prm[0]
    b1 = prm[1]
    psi0 = prm[2]
    psi1 = prm[3]
    sp = lambda s, p: p * jnp.log1p(jnp.exp(s / p))

    a0e = eA0[...]
    a1e = eA1[...]
    b0e = eB0[...]
    b1e = eB1[...]

    @pl.when(n == 0)
    def _():
        sc0 = a0e + b0e + b0
        sc1 = a1e + b1e + b1
        k0 = kd[...] == 0
        sck = jnp.where(k0, sc0, sc1)
        psik = jnp.where(k0, psi0, psi1)
        inten_o[...] = psik * jnp.log1p(jnp.exp(sck / psik))

    contrib = (sp(a0e + nB0[...] + b0, psi0)
               + sp(a1e + nB1[...] + b1, psi1)
               + sp(nA0[...] + b0e + b0, psi0)
               + sp(nA1[...] + b1e + b1, psi1)) * (1.0 / NNEG)

    @pl.when(n == 0)
    def _():
        surv_o[...] = contrib

    @pl.when(n > 0)
    def _():
        surv_o[...] += contrib


def _score_tc(a0, a1, bb0, bb1, kd, prm):
    neg_spec = pl.BlockSpec((1, BE), lambda n: (0, n))
    ev_spec = pl.BlockSpec((1, BE), lambda n: (0, NEG_BLKS))
    out_spec = pl.BlockSpec((1, BE), lambda n: (0, 0))
    out_shapes = (
        jax.ShapeDtypeStruct((1, B), jnp.float32),
        jax.ShapeDtypeStruct((1, B), jnp.float32),
    )
    return pl.pallas_call(
        _score_body,
        grid=(NNEG,),
        in_specs=[neg_spec, neg_spec, neg_spec, neg_spec,
                  ev_spec, ev_spec, ev_spec, ev_spec,
                  out_spec,
                  pl.BlockSpec(memory_space=pltpu.SMEM)],
        out_specs=(out_spec, out_spec),
        out_shape=out_shapes,
    )(a0, a1, bb0, bb1, a0, a1, bb0, bb1, kd, prm)


def kernel(table, W0, b0, W1, b1, psi, events, negs):
    v1 = events[:, 0].astype(jnp.int32)
    v2 = events[:, 1].astype(jnp.int32)
    kd = events[:, 4].astype(jnp.int32)[None, :]

    negT = jnp.transpose(negs.astype(jnp.int32), (1, 0, 2))
    idx_l = jnp.concatenate([negT[:, :, 0].reshape(-1), v1])
    idx_r = jnp.concatenate([negT[:, :, 1].reshape(-1), v2])

    tableT = jnp.swapaxes(table, 0, 1)
    p0, p1, p2, p3 = _project_tc(tableT, W0, W1)
    a0, a1, bb0, bb1 = _gather_proj_sc(p0, p1, p2, p3, idx_l, idx_r)

    prm = jnp.stack([b0[0], b1[0], psi[0, 0], psi[1, 0]])
    inten, surv = _score_tc(a0, a1, bb0, bb1, kd, prm)
    return inten, surv


# speedup vs baseline: 1.0783x; 1.0783x over previous
import functools

import jax
import jax.numpy as jnp
from jax import lax
from jax.experimental import pallas as pl
from jax.experimental.pallas import tpu as pltpu
from jax.experimental.pallas import tpu_sc as plsc

NSIZE = 1000000
EM = 64
B = 4096
NNEG = 20

NW = 32
RL = B * NNEG + B
PER_W = RL // NW
CHUNK = 2688
NCH = PER_W // CHUNK

PS = 65536
PGRID = (NSIZE + PS - 1) // PS

BE = B
NEG_BLKS = B * NNEG // BE


def _project_body(tblT, w8, o0, o1, o2, o3):
    P = lax.dot_general(w8[...], tblT[...], (((1,), (0,)), ((), ())),
                        preferred_element_type=jnp.float32)
    o0[...] = P[0, :]
    o1[...] = P[1, :]
    o2[...] = P[2, :]
    o3[...] = P[3, :]


def _project_tc(tableT, w8):
    out_shapes = tuple(
        jax.ShapeDtypeStruct((NSIZE,), jnp.float32) for _ in range(4))
    return pl.pallas_call(
        _project_body,
        grid=(PGRID,),
        in_specs=[
            pl.BlockSpec((EM, PS), lambda i: (0, i)),
            pl.BlockSpec((8, EM), lambda i: (0, 0)),
        ],
        out_specs=tuple(pl.BlockSpec((PS,), lambda i: (i,))
                        for _ in range(4)),
        out_shape=out_shapes,
    )(tableT, w8)


def _gather_proj_sc(p0, p1, p2, p3, idx_l, idx_r):
    mesh = plsc.VectorSubcoreMesh(core_axis_name="c", subcore_axis_name="s")

    @functools.partial(
        pl.kernel,
        out_type=tuple(
            jax.ShapeDtypeStruct((1, RL), jnp.float32) for _ in range(4)),
        mesh=mesh,
        compiler_params=pltpu.CompilerParams(use_tc_tiling_on_sc=False),
        scratch_types=[
            pltpu.VMEM((PER_W,), jnp.int32),
            pltpu.VMEM((PER_W,), jnp.int32),
            pltpu.VMEM((CHUNK,), jnp.float32),
            pltpu.VMEM((CHUNK,), jnp.float32),
            pltpu.VMEM((CHUNK,), jnp.float32),
            pltpu.VMEM((CHUNK,), jnp.float32),
            pltpu.SemaphoreType.DMA,
        ],
    )
    def gather_kernel(p0_h, p1_h, p2_h, p3_h, il_h, ir_h,
                      o0_h, o1_h, o2_h, o3_h,
                      il_v, ir_v, s0, s1, s2, s3, sem):
        wid = lax.axis_index("s") * 2 + lax.axis_index("c")
        base = wid * PER_W
        pltpu.sync_copy(il_h.at[pl.ds(base, PER_W)], il_v)
        pltpu.sync_copy(ir_h.at[pl.ds(base, PER_W)], ir_v)
        for c in range(NCH):
            il_c = il_v.at[pl.ds(c * CHUNK, CHUNK)]
            ir_c = ir_v.at[pl.ds(c * CHUNK, CHUNK)]
            cps = [
                pltpu.async_copy(p0_h.at[il_c], s0, sem),
                pltpu.async_copy(p1_h.at[il_c], s1, sem),
                pltpu.async_copy(p2_h.at[ir_c], s2, sem),
                pltpu.async_copy(p3_h.at[ir_c], s3, sem),
            ]
            for cp in cps:
                cp.wait()
            dst = pl.ds(base + c * CHUNK, CHUNK)
            pltpu.sync_copy(s0, o0_h.at[0, dst])
            pltpu.sync_copy(s1, o1_h.at[0, dst])
            pltpu.sync_copy(s2, o2_h.at[0, dst])
            pltpu.sync_copy(s3, o3_h.at[0, dst])

    return gather_kernel(p0, p1, p2, p3, idx_l, idx_r)


def _score_body(nA0, nA1, nB0, nB1, eA0, eA1, eB0, eB1, kd, prm,
                inten_o, surv_o):
    b0 = prm[0]
    b1 = prm[1]
    psi0 = prm[2]
    psi1 = prm[3]
    sp = lambda s, p: p * jnp.log1p(jnp.exp(s / p))

    a0e = eA0[...]
    a1e = eA1[...]
    b0e = eB0[...]
    b1e = eB1[...]

    sc0 = a0e + b0e + b0
    sc1 = a1e + b1e + b1
    k0 = kd[...] == 0
    sck = jnp.where(k0, sc0, sc1)
    psik = jnp.where(k0, psi0, psi1)
    inten_o[...] = psik * jnp.log1p(jnp.exp(sck / psik))

    acc = jnp.zeros((1, B), jnp.float32)
    for n in range(NNEG):
        s = pl.ds(n * B, B)
        acc += (sp(a0e + nB0[:, s] + b0, psi0)
                + sp(a1e + nB1[:, s] + b1, psi1)
                + sp(nA0[:, s] + b0e + b0, psi0)
                + sp(nA1[:, s] + b1e + b1, psi1))
    surv_o[...] = acc * (1.0 / NNEG)


def _score_tc(a0, a1, bb0, bb1, kd, prm):
    neg_spec = pl.BlockSpec((1, B * NNEG), lambda i: (0, 0))
    ev_spec = pl.BlockSpec((1, B), lambda i: (0, NNEG))
    out_spec = pl.BlockSpec((1, B), lambda i: (0, 0))
    out_shapes = (
        jax.ShapeDtypeStruct((1, B), jnp.float32),
        jax.ShapeDtypeStruct((1, B), jnp.float32),
    )
    return pl.pallas_call(
        _score_body,
        grid=(1,),
        in_specs=[neg_spec, neg_spec, neg_spec, neg_spec,
                  ev_spec, ev_spec, ev_spec, ev_spec,
                  out_spec,
                  pl.BlockSpec(memory_space=pltpu.SMEM)],
        out_specs=(out_spec, out_spec),
        out_shape=out_shapes,
    )(a0, a1, bb0, bb1, a0, a1, bb0, bb1, kd, prm)


def kernel(table, W0, b0, W1, b1, psi, events, negs):
    v1 = events[:, 0].astype(jnp.int32)
    v2 = events[:, 1].astype(jnp.int32)
    kd = events[:, 4].astype(jnp.int32)[None, :]

    negT = jnp.transpose(negs.astype(jnp.int32), (1, 0, 2))
    idx_l = jnp.concatenate([negT[:, :, 0].reshape(-1), v1])
    idx_r = jnp.concatenate([negT[:, :, 1].reshape(-1), v2])

    tableT = jnp.swapaxes(table, 0, 1)
    w0a = W0[:EM, 0]
    w0b = W0[EM:, 0]
    w1a = W1[:EM, 0]
    w1b = W1[EM:, 0]
    w8 = jnp.stack([w0a, w1a, w0b, w1b] + [jnp.zeros((EM,), jnp.float32)] * 4)

    p0, p1, p2, p3 = _project_tc(tableT, w8)
    a0, a1, bb0, bb1 = _gather_proj_sc(p0, p1, p2, p3, idx_l, idx_r)

    prm = jnp.stack([b0[0], b1[0], psi[0, 0], psi[1, 0]])
    inten, surv = _score_tc(a0, a1, bb0, bb1, kd, prm)
    return inten, surv


# speedup vs baseline: 1.0819x; 1.0034x over previous
import functools

import jax
import jax.numpy as jnp
from jax import lax
from jax.experimental import pallas as pl
from jax.experimental.pallas import tpu as pltpu
from jax.experimental.pallas import tpu_sc as plsc

NSIZE = 1000000
EM = 64
B = 4096
NNEG = 20

NW = 32
RL = B * NNEG + B
PER_W = RL // NW
CHUNK = 2688
NCH = PER_W // CHUNK

PS = 32768
PGRID = (NSIZE + PS - 1) // PS

BE = B
NEG_BLKS = B * NNEG // BE


def _project_body(tblT, w8, o0, o1, o2, o3):
    P = lax.dot_general(w8[...], tblT[...], (((1,), (0,)), ((), ())),
                        preferred_element_type=jnp.float32)
    o0[...] = P[0, :]
    o1[...] = P[1, :]
    o2[...] = P[2, :]
    o3[...] = P[3, :]


def _project_tc(tableT, w8):
    out_shapes = tuple(
        jax.ShapeDtypeStruct((NSIZE,), jnp.float32) for _ in range(4))
    return pl.pallas_call(
        _project_body,
        grid=(PGRID,),
        in_specs=[
            pl.BlockSpec((EM, PS), lambda i: (0, i)),
            pl.BlockSpec((8, EM), lambda i: (0, 0)),
        ],
        out_specs=tuple(pl.BlockSpec((PS,), lambda i: (i,))
                        for _ in range(4)),
        out_shape=out_shapes,
    )(tableT, w8)


def _gather_proj_sc(p0, p1, p2, p3, idx_l, idx_r):
    mesh = plsc.VectorSubcoreMesh(core_axis_name="c", subcore_axis_name="s")

    @functools.partial(
        pl.kernel,
        out_type=tuple(
            jax.ShapeDtypeStruct((1, RL), jnp.float32) for _ in range(4)),
        mesh=mesh,
        compiler_params=pltpu.CompilerParams(use_tc_tiling_on_sc=False),
        scratch_types=[
            pltpu.VMEM((PER_W,), jnp.int32),
            pltpu.VMEM((PER_W,), jnp.int32),
            pltpu.VMEM((CHUNK,), jnp.float32),
            pltpu.VMEM((CHUNK,), jnp.float32),
            pltpu.VMEM((CHUNK,), jnp.float32),
            pltpu.VMEM((CHUNK,), jnp.float32),
            pltpu.SemaphoreType.DMA,
        ],
    )
    def gather_kernel(p0_h, p1_h, p2_h, p3_h, il_h, ir_h,
                      o0_h, o1_h, o2_h, o3_h,
                      il_v, ir_v, s0, s1, s2, s3, sem):
        wid = lax.axis_index("s") * 2 + lax.axis_index("c")
        base = wid * PER_W
        pltpu.sync_copy(il_h.at[pl.ds(base, PER_W)], il_v)
        pltpu.sync_copy(ir_h.at[pl.ds(base, PER_W)], ir_v)
        for c in range(NCH):
            il_c = il_v.at[pl.ds(c * CHUNK, CHUNK)]
            ir_c = ir_v.at[pl.ds(c * CHUNK, CHUNK)]
            cps = [
                pltpu.async_copy(p0_h.at[il_c], s0, sem),
                pltpu.async_copy(p1_h.at[il_c], s1, sem),
                pltpu.async_copy(p2_h.at[ir_c], s2, sem),
                pltpu.async_copy(p3_h.at[ir_c], s3, sem),
            ]
            for cp in cps:
                cp.wait()
            dst = pl.ds(base + c * CHUNK, CHUNK)
            pltpu.sync_copy(s0, o0_h.at[0, dst])
            pltpu.sync_copy(s1, o1_h.at[0, dst])
            pltpu.sync_copy(s2, o2_h.at[0, dst])
            pltpu.sync_copy(s3, o3_h.at[0, dst])

    return gather_kernel(p0, p1, p2, p3, idx_l, idx_r)


def _score_body(nA0, nA1, nB0, nB1, eA0, eA1, eB0, eB1, kd, prm,
                inten_o, surv_o):
    b0 = prm[0]
    b1 = prm[1]
    psi0 = prm[2]
    psi1 = prm[3]
    sp = lambda s, p: p * jnp.log1p(jnp.exp(s / p))

    a0e = eA0[...]
    a1e = eA1[...]
    b0e = eB0[...]
    b1e = eB1[...]

    sc0 = a0e + b0e + b0
    sc1 = a1e + b1e + b1
    k0 = kd[...] == 0
    sck = jnp.where(k0, sc0, sc1)
    psik = jnp.where(k0, psi0, psi1)
    inten_o[...] = psik * jnp.log1p(jnp.exp(sck / psik))

    acc = jnp.zeros((1, B), jnp.float32)
    for n in range(NNEG):
        s = pl.ds(n * B, B)
        acc += (sp(a0e + nB0[:, s] + b0, psi0)
                + sp(a1e + nB1[:, s] + b1, psi1)
                + sp(nA0[:, s] + b0e + b0, psi0)
                + sp(nA1[:, s] + b1e + b1, psi1))
    surv_o[...] = acc * (1.0 / NNEG)


def _score_tc(a0, a1, bb0, bb1, kd, prm):
    neg_spec = pl.BlockSpec((1, B * NNEG), lambda i: (0, 0))
    ev_spec = pl.BlockSpec((1, B), lambda i: (0, NNEG))
    out_spec = pl.BlockSpec((1, B), lambda i: (0, 0))
    out_shapes = (
        jax.ShapeDtypeStruct((1, B), jnp.float32),
        jax.ShapeDtypeStruct((1, B), jnp.float32),
    )
    return pl.pallas_call(
        _score_body,
        grid=(1,),
        in_specs=[neg_spec, neg_spec, neg_spec, neg_spec,
                  ev_spec, ev_spec, ev_spec, ev_spec,
                  out_spec,
                  pl.BlockSpec(memory_space=pltpu.SMEM)],
        out_specs=(out_spec, out_spec),
        out_shape=out_shapes,
    )(a0, a1, bb0, bb1, a0, a1, bb0, bb1, kd, prm)


def kernel(table, W0, b0, W1, b1, psi, events, negs):
    v1 = events[:, 0].astype(jnp.int32)
    v2 = events[:, 1].astype(jnp.int32)
    kd = events[:, 4].astype(jnp.int32)[None, :]

    negT = jnp.transpose(negs.astype(jnp.int32), (1, 0, 2))
    idx_l = jnp.concatenate([negT[:, :, 0].reshape(-1), v1])
    idx_r = jnp.concatenate([negT[:, :, 1].reshape(-1), v2])

    tableT = jnp.swapaxes(table, 0, 1)
    w0a = W0[:EM, 0]
    w0b = W0[EM:, 0]
    w1a = W1[:EM, 0]
    w1b = W1[EM:, 0]
    w8 = jnp.stack([w0a, w1a, w0b, w1b] + [jnp.zeros((EM,), jnp.float32)] * 4)

    p0, p1, p2, p3 = _project_tc(tableT, w8)
    a0, a1, bb0, bb1 = _gather_proj_sc(p0, p1, p2, p3, idx_l, idx_r)

    prm = jnp.stack([b0[0], b1[0], psi[0, 0], psi[1, 0]])
    inten, surv = _score_tc(a0, a1, bb0, bb1, kd, prm)
    return inten, surv
